# baseline (speedup 1.0000x reference)
import jax
import jax.numpy as jnp
from jax import lax
from jax.experimental import pallas as pl
from jax.experimental.pallas import tpu as pltpu
from jax.experimental.pallas import tpu_sc as plsc

N_NODES = 10000
N_EDGES = 320000
NODE_DIM = 128
MAX_DEG = 512

NC = 2
NS = 16
L = 16
NW = NC * NS

HROWS = 80
HCOLS = 128
EPT = N_EDGES // NS
ECHUNK = 10000
SC_UNROLL = 5

CHUNK = 128
NFULL = N_NODES // CHUNK
KMAX = (NFULL + NW - 1) // NW
TBASE = NFULL * CHUNK
TAIL = N_NODES - TBASE
ROW_UNROLL = 4

_mesh = plsc.VectorSubcoreMesh(core_axis_name="c", subcore_axis_name="s",
                               num_cores=NC, num_subcores=NS)


def _fused_kernel(x_hbm, src_hbm, dst_hbm, zin_hbm, zout_hbm, out_hbm,
                  ebuf0_v, ebuf1_v, hin_v, hout_v, idx80_v,
                  di0_v, di1_v, di2_v, do0_v, do1_v, do2_v,
                  xacc_v, a_v, b_v,
                  zin_sh, zout_sh, degin_sh, degout_sh,
                  sem_e, sem_t, sem_a, sem_b, sem_o,
                  sem_x0, sem_x1, sem_x2):
    cid = lax.axis_index("c")
    sid = lax.axis_index("s")
    wid = sid * NC + cid
    di_refs = (di0_v, di1_v, di2_v)
    do_refs = (do0_v, do1_v, do2_v)
    xsems = (sem_x0, sem_x1, sem_x2)

    trows = MAX_DEG // NS
    tr = pl.ds(sid * trows, trows)
    ct1 = pltpu.async_copy(zin_hbm.at[tr], zin_sh.at[tr], sem_t)
    ct2 = pltpu.async_copy(zout_hbm.at[tr], zout_sh.at[tr], sem_t)

    for k in range(2):
        ck = wid + k * NW

        @pl.when(ck < NFULL)
        def _():
            pltpu.async_copy(x_hbm.at[pl.ds(ck * CHUNK, CHUNK)],
                             xacc_v.at[k], xsems[k])

    ebase = sid * EPT
    ce = pltpu.async_copy(dst_hbm.at[pl.ds(ebase, ECHUNK)], ebuf0_v, sem_e)

    zeros = jnp.zeros((L,), jnp.int32)

    def zero_body(i, c):
        for u in range(4):
            r = i * 4 + u
            for j in range(HCOLS // L):
                s = pl.ds(j * L, L)
                hin_v[r, s] = zeros
                hout_v[r, s] = zeros
        return c
    lax.fori_loop(0, HROWS // 4, zero_body, 0)

    iota = lax.iota(jnp.int32, L)
    for i in range(HROWS // L):
        idx80_v[pl.ds(i * L, L)] = iota + (i * L)

    @pl.when(sid == 0)
    def _():
        pltpu.sync_copy(hin_v, degin_sh)
        pltpu.sync_copy(hout_v, degout_sh)
    plsc.subcore_barrier()

    ones = jnp.ones((L,), jnp.int32)

    def scatter_chunk(ebuf, hist):
        def scat_body(i, c):
            for u in range(SC_UNROLL):
                idx = ebuf[pl.ds((i * SC_UNROLL + u) * L, L)]
                plsc.addupdate_scatter(
                    hist,
                    [lax.shift_right_logical(idx, 7),
                     lax.bitwise_and(idx, 127)],
                    ones)
            return c
        lax.fori_loop(0, ECHUNK // (L * SC_UNROLL), scat_body, 0)

    ce.wait()
    ce = pltpu.async_copy(dst_hbm.at[pl.ds(ebase + ECHUNK, ECHUNK)],
                          ebuf1_v, sem_e)
    scatter_chunk(ebuf0_v, hin_v)
    ce.wait()
    ce = pltpu.async_copy(src_hbm.at[pl.ds(ebase, ECHUNK)], ebuf0_v, sem_e)
    scatter_chunk(ebuf1_v, hin_v)
    ce.wait()
    ce = pltpu.async_copy(src_hbm.at[pl.ds(ebase + ECHUNK, ECHUNK)],
                          ebuf1_v, sem_e)
    scatter_chunk(ebuf0_v, hout_v)
    ce.wait()
    scatter_chunk(ebuf1_v, hout_v)

    pltpu.sync_copy(hin_v, degin_sh.at[idx80_v], add=True)
    pltpu.sync_copy(hout_v, degout_sh.at[idx80_v], add=True)
    ct1.wait()
    ct2.wait()
    plsc.subcore_barrier()

    cap = jnp.full((L,), MAX_DEG - 1, jnp.int32)

    def load_deg(row, di, do):
        pltpu.sync_copy(degin_sh.at[row], di)
        pltpu.sync_copy(degout_sh.at[row], do)
        for j in range(CHUNK // L):
            s = pl.ds(j * L, L)
            di[s] = jnp.minimum(di[s], cap)
            do[s] = jnp.minimum(do[s], cap)

    def add_rows(k, nrows):
        def add_body(i, c):
            for u in range(ROW_UNROLL):
                r = i * ROW_UNROLL + u
                for j in range(NODE_DIM // L):
                    s = pl.ds(j * L, L)
                    xacc_v[k, r, s] = (xacc_v[k, r, s]
                                       + a_v[r, s] + b_v[r, s])
            return c
        lax.fori_loop(0, nrows // ROW_UNROLL, add_body, 0)

    for k in range(KMAX):
        ck = wid + k * NW
        slot = k % 2

        @pl.when(ck < NFULL)
        def _():
            nb = ck * CHUNK
            load_deg(ck, di_refs[k], do_refs[k])
            ga = pltpu.async_copy(zin_sh.at[di_refs[k]], a_v, sem_a)
            gb = pltpu.async_copy(zout_sh.at[do_refs[k]], b_v, sem_b)
            if k == 2:
                pltpu.make_async_copy(
                    xacc_v.at[0], out_hbm.at[pl.ds(wid * CHUNK, CHUNK)],
                    sem_o).wait()
                pltpu.async_copy(x_hbm.at[pl.ds(nb, CHUNK)],
                                 xacc_v.at[slot], xsems[0])
            pltpu.make_async_copy(x_hbm.at[pl.ds(nb, CHUNK)],
                                  xacc_v.at[slot], xsems[0 if k == 2 else k]
                                  ).wait()
            ga.wait()
            gb.wait()
            add_rows(slot, CHUNK)
            pltpu.async_copy(xacc_v.at[slot], out_hbm.at[pl.ds(nb, CHUNK)],
                             sem_o)

    @pl.when(wid == NW - 1)
    def _():
        load_deg(NFULL, di_refs[2], do_refs[2])
        ga = pltpu.async_copy(zin_sh.at[di_refs[2]], a_v, sem_a)
        gb = pltpu.async_copy(zout_sh.at[do_refs[2]], b_v, sem_b)
        pltpu.make_async_copy(
            xacc_v.at[0], out_hbm.at[pl.ds(wid * CHUNK, CHUNK)],
            sem_o).wait()
        pltpu.async_copy(x_hbm.at[pl.ds(TBASE, TAIL)],
                         xacc_v.at[0, pl.ds(0, TAIL)], sem_x2)
        pltpu.make_async_copy(x_hbm.at[pl.ds(TBASE, TAIL)],
                              xacc_v.at[0, pl.ds(0, TAIL)], sem_x2).wait()
        ga.wait()
        gb.wait()
        for i in range(TAIL):
            for j in range(NODE_DIM // L):
                s = pl.ds(j * L, L)
                xacc_v[0, i, s] = xacc_v[0, i, s] + a_v[i, s] + b_v[i, s]
        pltpu.sync_copy(xacc_v.at[0, pl.ds(0, TAIL)],
                        out_hbm.at[pl.ds(TBASE, TAIL)])

    skip0 = ((wid + 2 * NW) < NFULL) | (wid == NW - 1)
    for k in range(KMAX):
        ck = wid + k * NW
        cond = (ck < NFULL) & (~skip0) if k == 0 else (ck < NFULL)

        @pl.when(cond)
        def _():
            pltpu.make_async_copy(
                xacc_v.at[k % 2], out_hbm.at[pl.ds(ck * CHUNK, CHUNK)],
                sem_o).wait()


def kernel(x, edge_index, z_in, z_out):
    edge_index = edge_index.astype(jnp.int32)
    src = edge_index[0]
    dst = edge_index[1]

    call = pl.kernel(
        _fused_kernel,
        out_type=jax.ShapeDtypeStruct((N_NODES, NODE_DIM), jnp.float32),
        mesh=_mesh,
        scratch_types=[
            pltpu.VMEM((ECHUNK,), jnp.int32),
            pltpu.VMEM((ECHUNK,), jnp.int32),
            pltpu.VMEM((HROWS, HCOLS), jnp.int32),
            pltpu.VMEM((HROWS, HCOLS), jnp.int32),
            pltpu.VMEM((HROWS,), jnp.int32),
            pltpu.VMEM((CHUNK,), jnp.int32),
            pltpu.VMEM((CHUNK,), jnp.int32),
            pltpu.VMEM((CHUNK,), jnp.int32),
            pltpu.VMEM((CHUNK,), jnp.int32),
            pltpu.VMEM((CHUNK,), jnp.int32),
            pltpu.VMEM((CHUNK,), jnp.int32),
            pltpu.VMEM((2, CHUNK, NODE_DIM), jnp.float32),
            pltpu.VMEM((CHUNK, NODE_DIM), jnp.float32),
            pltpu.VMEM((CHUNK, NODE_DIM), jnp.float32),
            pltpu.VMEM_SHARED((MAX_DEG, NODE_DIM), jnp.float32),
            pltpu.VMEM_SHARED((MAX_DEG, NODE_DIM), jnp.float32),
            pltpu.VMEM_SHARED((HROWS, HCOLS), jnp.int32),
            pltpu.VMEM_SHARED((HROWS, HCOLS), jnp.int32),
            pltpu.SemaphoreType.DMA,
            pltpu.SemaphoreType.DMA,
            pltpu.SemaphoreType.DMA,
            pltpu.SemaphoreType.DMA,
            pltpu.SemaphoreType.DMA,
            pltpu.SemaphoreType.DMA,
            pltpu.SemaphoreType.DMA,
            pltpu.SemaphoreType.DMA,
        ],
        compiler_params=pltpu.CompilerParams(needs_layout_passes=False),
    )
    return call(x, src, dst, z_in, z_out)


# speedup vs baseline: 1.2055x; 1.2055x over previous
import jax
import jax.numpy as jnp
from jax import lax
from jax.experimental import pallas as pl
from jax.experimental.pallas import tpu as pltpu
from jax.experimental.pallas import tpu_sc as plsc

N_NODES = 10000
N_EDGES = 320000
NODE_DIM = 128
MAX_DEG = 512

NC = 2
NS = 16
L = 16
NW = NC * NS

HROWS = 80
HCOLS = 128
EPT = N_EDGES // NS
ECHUNK = 10000
SC_UNROLL = 5

CHUNK = 128
NFULL = N_NODES // CHUNK
KMAX = (NFULL + NW - 1) // NW
TBASE = NFULL * CHUNK
TAIL = N_NODES - TBASE
ROW_UNROLL = 4

_mesh = plsc.VectorSubcoreMesh(core_axis_name="c", subcore_axis_name="s",
                               num_cores=NC, num_subcores=NS)


def _fused_kernel(x_hbm, src_hbm, dst_hbm, zin_hbm, zout_hbm, out_hbm,
                  ebuf0_v, ebuf1_v, hin_v, hout_v, idx80_v,
                  di0_v, di1_v, di2_v, do0_v, do1_v, do2_v,
                  xacc_v, a_v, b_v,
                  zin_sh, zout_sh, degin_sh, degout_sh,
                  sem_e, sem_t, sem_a, sem_b, sem_o,
                  sem_x0, sem_x1, sem_x2):
    cid = lax.axis_index("c")
    sid = lax.axis_index("s")
    wid = sid * NC + cid
    di_refs = (di0_v, di1_v, di2_v)
    do_refs = (do0_v, do1_v, do2_v)
    xsems = (sem_x0, sem_x1, sem_x2)

    trows = MAX_DEG // NS
    tr = pl.ds(sid * trows, trows)
    ct1 = pltpu.async_copy(zin_hbm.at[tr], zin_sh.at[tr], sem_t)
    ct2 = pltpu.async_copy(zout_hbm.at[tr], zout_sh.at[tr], sem_t)

    for k in range(2):
        ck = wid + k * NW

        @pl.when(ck < NFULL)
        def _():
            pltpu.async_copy(x_hbm.at[pl.ds(ck * CHUNK, CHUNK)],
                             xacc_v.at[k], xsems[k])

    ebase = sid * EPT
    ce = pltpu.async_copy(dst_hbm.at[pl.ds(ebase, ECHUNK)], ebuf0_v, sem_e)

    zeros = jnp.zeros((L,), jnp.int32)

    @plsc.parallel_loop(0, HROWS, unroll=4)
    def _(r):
        for j in range(HCOLS // L):
            s = pl.ds(j * L, L)
            hin_v[r, s] = zeros
            hout_v[r, s] = zeros

    iota = lax.iota(jnp.int32, L)
    for i in range(HROWS // L):
        idx80_v[pl.ds(i * L, L)] = iota + (i * L)

    @pl.when(sid == 0)
    def _():
        pltpu.sync_copy(hin_v, degin_sh)
        pltpu.sync_copy(hout_v, degout_sh)
    plsc.subcore_barrier()

    ones = jnp.ones((L,), jnp.int32)

    def scatter_chunk(ebuf, hist):
        @plsc.parallel_loop(0, ECHUNK, step=L, unroll=SC_UNROLL)
        def _(i):
            idx = ebuf[pl.ds(i, L)]
            plsc.addupdate_scatter(
                hist,
                [lax.shift_right_logical(idx, 7),
                 lax.bitwise_and(idx, 127)],
                ones)

    ce.wait()
    ce = pltpu.async_copy(dst_hbm.at[pl.ds(ebase + ECHUNK, ECHUNK)],
                          ebuf1_v, sem_e)
    scatter_chunk(ebuf0_v, hin_v)
    ce.wait()
    ce = pltpu.async_copy(src_hbm.at[pl.ds(ebase, ECHUNK)], ebuf0_v, sem_e)
    scatter_chunk(ebuf1_v, hin_v)
    ce.wait()
    ce = pltpu.async_copy(src_hbm.at[pl.ds(ebase + ECHUNK, ECHUNK)],
                          ebuf1_v, sem_e)
    scatter_chunk(ebuf0_v, hout_v)
    ce.wait()
    scatter_chunk(ebuf1_v, hout_v)

    pltpu.sync_copy(hin_v, degin_sh.at[idx80_v], add=True)
    pltpu.sync_copy(hout_v, degout_sh.at[idx80_v], add=True)
    ct1.wait()
    ct2.wait()
    plsc.subcore_barrier()

    cap = jnp.full((L,), MAX_DEG - 1, jnp.int32)

    def load_deg(row, di, do):
        pltpu.sync_copy(degin_sh.at[row], di)
        pltpu.sync_copy(degout_sh.at[row], do)
        for j in range(CHUNK // L):
            s = pl.ds(j * L, L)
            di[s] = jnp.minimum(di[s], cap)
            do[s] = jnp.minimum(do[s], cap)

    def add_rows(k, nrows):
        @plsc.parallel_loop(0, nrows, unroll=ROW_UNROLL)
        def _(r):
            for j in range(NODE_DIM // L):
                s = pl.ds(j * L, L)
                xacc_v[k, r, s] = (xacc_v[k, r, s]
                                   + a_v[r, s] + b_v[r, s])

    for k in range(KMAX):
        ck = wid + k * NW
        slot = k % 2

        @pl.when(ck < NFULL)
        def _():
            nb = ck * CHUNK
            load_deg(ck, di_refs[k], do_refs[k])
            ga = pltpu.async_copy(zin_sh.at[di_refs[k]], a_v, sem_a)
            gb = pltpu.async_copy(zout_sh.at[do_refs[k]], b_v, sem_b)
            if k == 2:
                pltpu.make_async_copy(
                    xacc_v.at[0], out_hbm.at[pl.ds(wid * CHUNK, CHUNK)],
                    sem_o).wait()
                pltpu.async_copy(x_hbm.at[pl.ds(nb, CHUNK)],
                                 xacc_v.at[slot], xsems[0])
            pltpu.make_async_copy(x_hbm.at[pl.ds(nb, CHUNK)],
                                  xacc_v.at[slot], xsems[0 if k == 2 else k]
                                  ).wait()
            ga.wait()
            gb.wait()
            add_rows(slot, CHUNK)
            pltpu.async_copy(xacc_v.at[slot], out_hbm.at[pl.ds(nb, CHUNK)],
                             sem_o)

    @pl.when(wid == NW - 1)
    def _():
        load_deg(NFULL, di_refs[2], do_refs[2])
        ga = pltpu.async_copy(zin_sh.at[di_refs[2]], a_v, sem_a)
        gb = pltpu.async_copy(zout_sh.at[do_refs[2]], b_v, sem_b)
        pltpu.make_async_copy(
            xacc_v.at[0], out_hbm.at[pl.ds(wid * CHUNK, CHUNK)],
            sem_o).wait()
        pltpu.async_copy(x_hbm.at[pl.ds(TBASE, TAIL)],
                         xacc_v.at[0, pl.ds(0, TAIL)], sem_x2)
        pltpu.make_async_copy(x_hbm.at[pl.ds(TBASE, TAIL)],
                              xacc_v.at[0, pl.ds(0, TAIL)], sem_x2).wait()
        ga.wait()
        gb.wait()
        for i in range(TAIL):
            for j in range(NODE_DIM // L):
                s = pl.ds(j * L, L)
                xacc_v[0, i, s] = xacc_v[0, i, s] + a_v[i, s] + b_v[i, s]
        pltpu.sync_copy(xacc_v.at[0, pl.ds(0, TAIL)],
                        out_hbm.at[pl.ds(TBASE, TAIL)])

    skip0 = ((wid + 2 * NW) < NFULL) | (wid == NW - 1)
    for k in range(KMAX):
        ck = wid + k * NW
        cond = (ck < NFULL) & (~skip0) if k == 0 else (ck < NFULL)

        @pl.when(cond)
        def _():
            pltpu.make_async_copy(
                xacc_v.at[k % 2], out_hbm.at[pl.ds(ck * CHUNK, CHUNK)],
                sem_o).wait()


def kernel(x, edge_index, z_in, z_out):
    edge_index = edge_index.astype(jnp.int32)
    src = edge_index[0]
    dst = edge_index[1]

    call = pl.kernel(
        _fused_kernel,
        out_type=jax.ShapeDtypeStruct((N_NODES, NODE_DIM), jnp.float32),
        mesh=_mesh,
        scratch_types=[
            pltpu.VMEM((ECHUNK,), jnp.int32),
            pltpu.VMEM((ECHUNK,), jnp.int32),
            pltpu.VMEM((HROWS, HCOLS), jnp.int32),
            pltpu.VMEM((HROWS, HCOLS), jnp.int32),
            pltpu.VMEM((HROWS,), jnp.int32),
            pltpu.VMEM((CHUNK,), jnp.int32),
            pltpu.VMEM((CHUNK,), jnp.int32),
            pltpu.VMEM((CHUNK,), jnp.int32),
            pltpu.VMEM((CHUNK,), jnp.int32),
            pltpu.VMEM((CHUNK,), jnp.int32),
            pltpu.VMEM((CHUNK,), jnp.int32),
            pltpu.VMEM((2, CHUNK, NODE_DIM), jnp.float32),
            pltpu.VMEM((CHUNK, NODE_DIM), jnp.float32),
            pltpu.VMEM((CHUNK, NODE_DIM), jnp.float32),
            pltpu.VMEM_SHARED((MAX_DEG, NODE_DIM), jnp.float32),
            pltpu.VMEM_SHARED((MAX_DEG, NODE_DIM), jnp.float32),
            pltpu.VMEM_SHARED((HROWS, HCOLS), jnp.int32),
            pltpu.VMEM_SHARED((HROWS, HCOLS), jnp.int32),
            pltpu.SemaphoreType.DMA,
            pltpu.SemaphoreType.DMA,
            pltpu.SemaphoreType.DMA,
            pltpu.SemaphoreType.DMA,
            pltpu.SemaphoreType.DMA,
            pltpu.SemaphoreType.DMA,
            pltpu.SemaphoreType.DMA,
            pltpu.SemaphoreType.DMA,
        ],
        compiler_params=pltpu.CompilerParams(needs_layout_passes=False),
    )
    return call(x, src, dst, z_in, z_out)


# speedup vs baseline: 1.4869x; 1.2334x over previous
import jax
import jax.numpy as jnp
from jax import lax
from jax.experimental import pallas as pl
from jax.experimental.pallas import tpu as pltpu
from jax.experimental.pallas import tpu_sc as plsc

N_NODES = 10000
N_EDGES = 320000
NODE_DIM = 128
MAX_DEG = 512

NC = 2
NS = 16
L = 16
NW = NC * NS

HROWS = 80
HCOLS = 128
EPT = N_EDGES // NS
ECHUNK = 10000
SC_UNROLL = 5

CHUNK = 128
NFULL = N_NODES // CHUNK
KMAX = (NFULL + NW - 1) // NW
TBASE = NFULL * CHUNK
TAIL = N_NODES - TBASE
ROW_UNROLL = 4

_mesh = plsc.VectorSubcoreMesh(core_axis_name="c", subcore_axis_name="s",
                               num_cores=NC, num_subcores=NS)


def _fused_kernel(x_hbm, edges_hbm, zin_hbm, zout_hbm, out_hbm,
                  ebuf0_v, ebuf1_v, hin_v, hout_v, idx80_v,
                  di0_v, di1_v, di2_v, do0_v, do1_v, do2_v,
                  xacc_v, a_v, b_v,
                  zin_sh, zout_sh, degin_sh, degout_sh,
                  sem_e, sem_t, sem_a, sem_b, sem_o,
                  sem_x0, sem_x1, sem_x2):
    cid = lax.axis_index("c")
    sid = lax.axis_index("s")
    wid = sid * NC + cid
    di_refs = (di0_v, di1_v, di2_v)
    do_refs = (do0_v, do1_v, do2_v)
    xsems = (sem_x0, sem_x1, sem_x2)

    trows = MAX_DEG // NS
    tr = pl.ds(sid * trows, trows)
    ct1 = pltpu.async_copy(zin_hbm.at[tr], zin_sh.at[tr], sem_t)
    ct2 = pltpu.async_copy(zout_hbm.at[tr], zout_sh.at[tr], sem_t)

    for k in range(2):
        ck = wid + k * NW

        @pl.when(ck < NFULL)
        def _():
            pltpu.async_copy(x_hbm.at[pl.ds(ck * CHUNK, CHUNK)],
                             xacc_v.at[k], xsems[k])

    ebase = sid * EPT
    ce = pltpu.async_copy(edges_hbm.at[pl.ds(N_EDGES + ebase, ECHUNK)],
                          ebuf0_v, sem_e)

    zeros = jnp.zeros((L,), jnp.int32)

    @plsc.parallel_loop(0, HROWS, unroll=4)
    def _(r):
        for j in range(HCOLS // L):
            s = pl.ds(j * L, L)
            hin_v[r, s] = zeros
            hout_v[r, s] = zeros

    iota = lax.iota(jnp.int32, L)
    for i in range(HROWS // L):
        idx80_v[pl.ds(i * L, L)] = iota + (i * L)

    @pl.when(sid == 0)
    def _():
        pltpu.sync_copy(hin_v, degin_sh)
        pltpu.sync_copy(hout_v, degout_sh)
    plsc.subcore_barrier()

    ones = jnp.ones((L,), jnp.int32)

    def scatter_chunk(ebuf, hist):
        @plsc.parallel_loop(0, ECHUNK, step=L, unroll=SC_UNROLL)
        def _(i):
            idx = ebuf[pl.ds(i, L)]
            plsc.addupdate_scatter(
                hist,
                [lax.shift_right_logical(idx, 7),
                 lax.bitwise_and(idx, 127)],
                ones)

    ce.wait()
    ce = pltpu.async_copy(
        edges_hbm.at[pl.ds(N_EDGES + ebase + ECHUNK, ECHUNK)], ebuf1_v,
        sem_e)
    scatter_chunk(ebuf0_v, hin_v)
    ce.wait()
    ce = pltpu.async_copy(edges_hbm.at[pl.ds(ebase, ECHUNK)], ebuf0_v,
                          sem_e)
    scatter_chunk(ebuf1_v, hin_v)
    ce.wait()
    ce = pltpu.async_copy(edges_hbm.at[pl.ds(ebase + ECHUNK, ECHUNK)],
                          ebuf1_v, sem_e)
    scatter_chunk(ebuf0_v, hout_v)
    ce.wait()
    scatter_chunk(ebuf1_v, hout_v)

    pltpu.sync_copy(hin_v, degin_sh.at[idx80_v], add=True)
    pltpu.sync_copy(hout_v, degout_sh.at[idx80_v], add=True)
    ct1.wait()
    ct2.wait()
    plsc.subcore_barrier()

    cap = jnp.full((L,), MAX_DEG - 1, jnp.int32)

    def load_deg(row, di, do):
        pltpu.sync_copy(degin_sh.at[row], di)
        pltpu.sync_copy(degout_sh.at[row], do)
        for j in range(CHUNK // L):
            s = pl.ds(j * L, L)
            di[s] = jnp.minimum(di[s], cap)
            do[s] = jnp.minimum(do[s], cap)

    def add_rows(k, nrows):
        @plsc.parallel_loop(0, nrows, unroll=ROW_UNROLL)
        def _(r):
            for j in range(NODE_DIM // L):
                s = pl.ds(j * L, L)
                xacc_v[k, r, s] = (xacc_v[k, r, s]
                                   + a_v[r, s] + b_v[r, s])

    for k in range(KMAX):
        ck = wid + k * NW
        slot = k % 2

        @pl.when(ck < NFULL)
        def _():
            nb = ck * CHUNK
            load_deg(ck, di_refs[k], do_refs[k])
            ga = pltpu.async_copy(zin_sh.at[di_refs[k]], a_v, sem_a)
            gb = pltpu.async_copy(zout_sh.at[do_refs[k]], b_v, sem_b)
            if k == 2:
                pltpu.make_async_copy(
                    xacc_v.at[0], out_hbm.at[pl.ds(wid * CHUNK, CHUNK)],
                    sem_o).wait()
                pltpu.async_copy(x_hbm.at[pl.ds(nb, CHUNK)],
                                 xacc_v.at[slot], xsems[0])
            pltpu.make_async_copy(x_hbm.at[pl.ds(nb, CHUNK)],
                                  xacc_v.at[slot], xsems[0 if k == 2 else k]
                                  ).wait()
            ga.wait()
            gb.wait()
            add_rows(slot, CHUNK)
            pltpu.async_copy(xacc_v.at[slot], out_hbm.at[pl.ds(nb, CHUNK)],
                             sem_o)

    @pl.when(wid == NW - 1)
    def _():
        load_deg(NFULL, di_refs[2], do_refs[2])
        ga = pltpu.async_copy(zin_sh.at[di_refs[2]], a_v, sem_a)
        gb = pltpu.async_copy(zout_sh.at[do_refs[2]], b_v, sem_b)
        pltpu.make_async_copy(
            xacc_v.at[0], out_hbm.at[pl.ds(wid * CHUNK, CHUNK)],
            sem_o).wait()
        pltpu.async_copy(x_hbm.at[pl.ds(TBASE, TAIL)],
                         xacc_v.at[0, pl.ds(0, TAIL)], sem_x2)
        pltpu.make_async_copy(x_hbm.at[pl.ds(TBASE, TAIL)],
                              xacc_v.at[0, pl.ds(0, TAIL)], sem_x2).wait()
        ga.wait()
        gb.wait()
        for i in range(TAIL):
            for j in range(NODE_DIM // L):
                s = pl.ds(j * L, L)
                xacc_v[0, i, s] = xacc_v[0, i, s] + a_v[i, s] + b_v[i, s]
        pltpu.sync_copy(xacc_v.at[0, pl.ds(0, TAIL)],
                        out_hbm.at[pl.ds(TBASE, TAIL)])

    skip0 = ((wid + 2 * NW) < NFULL) | (wid == NW - 1)
    for k in range(KMAX):
        ck = wid + k * NW
        cond = (ck < NFULL) & (~skip0) if k == 0 else (ck < NFULL)

        @pl.when(cond)
        def _():
            pltpu.make_async_copy(
                xacc_v.at[k % 2], out_hbm.at[pl.ds(ck * CHUNK, CHUNK)],
                sem_o).wait()


def kernel(x, edge_index, z_in, z_out):
    edges = edge_index.astype(jnp.int32).reshape(-1)

    call = pl.kernel(
        _fused_kernel,
        out_type=jax.ShapeDtypeStruct((N_NODES, NODE_DIM), jnp.float32),
        mesh=_mesh,
        scratch_types=[
            pltpu.VMEM((ECHUNK,), jnp.int32),
            pltpu.VMEM((ECHUNK,), jnp.int32),
            pltpu.VMEM((HROWS, HCOLS), jnp.int32),
            pltpu.VMEM((HROWS, HCOLS), jnp.int32),
            pltpu.VMEM((HROWS,), jnp.int32),
            pltpu.VMEM((CHUNK,), jnp.int32),
            pltpu.VMEM((CHUNK,), jnp.int32),
            pltpu.VMEM((CHUNK,), jnp.int32),
            pltpu.VMEM((CHUNK,), jnp.int32),
            pltpu.VMEM((CHUNK,), jnp.int32),
            pltpu.VMEM((CHUNK,), jnp.int32),
            pltpu.VMEM((2, CHUNK, NODE_DIM), jnp.float32),
            pltpu.VMEM((CHUNK, NODE_DIM), jnp.float32),
            pltpu.VMEM((CHUNK, NODE_DIM), jnp.float32),
            pltpu.VMEM_SHARED((MAX_DEG, NODE_DIM), jnp.float32),
            pltpu.VMEM_SHARED((MAX_DEG, NODE_DIM), jnp.float32),
            pltpu.VMEM_SHARED((HROWS, HCOLS), jnp.int32),
            pltpu.VMEM_SHARED((HROWS, HCOLS), jnp.int32),
            pltpu.SemaphoreType.DMA,
            pltpu.SemaphoreType.DMA,
            pltpu.SemaphoreType.DMA,
            pltpu.SemaphoreType.DMA,
            pltpu.SemaphoreType.DMA,
            pltpu.SemaphoreType.DMA,
            pltpu.SemaphoreType.DMA,
            pltpu.SemaphoreType.DMA,
        ],
        compiler_params=pltpu.CompilerParams(needs_layout_passes=False),
    )
    return call(x, edges, z_in, z_out)


# speedup vs baseline: 1.5004x; 1.0091x over previous
import jax
import jax.numpy as jnp
from jax import lax
from jax.experimental import pallas as pl
from jax.experimental.pallas import tpu as pltpu
from jax.experimental.pallas import tpu_sc as plsc

N_NODES = 10000
N_EDGES = 320000
NODE_DIM = 128
MAX_DEG = 512

NC = 2
NS = 16
L = 16
NW = NC * NS

HROWS = 80
HCOLS = 128
EPT = N_EDGES // NS
ECHUNK = 10000
SC_UNROLL = 5

CHUNK = 128
NFULL = N_NODES // CHUNK
KMAX = (NFULL + NW - 1) // NW
TBASE = NFULL * CHUNK
TAIL = N_NODES - TBASE
ROW_UNROLL = 4

_mesh = plsc.VectorSubcoreMesh(core_axis_name="c", subcore_axis_name="s",
                               num_cores=NC, num_subcores=NS)


def _fused_kernel(x_hbm, edges_hbm, zin_hbm, zout_hbm, out_hbm,
                  ebuf0_v, ebuf1_v, hin_v, hout_v, idx80_v,
                  di0_v, di1_v, di2_v, do0_v, do1_v, do2_v,
                  xacc_v, a_v, b_v,
                  zin_sh, zout_sh, degin_sh, degout_sh,
                  sem_e, sem_t, sem_a, sem_b, sem_o,
                  sem_x0, sem_x1, sem_x2):
    cid = lax.axis_index("c")
    sid = lax.axis_index("s")
    wid = sid * NC + cid
    di_refs = (di0_v, di1_v, di2_v)
    do_refs = (do0_v, do1_v, do2_v)
    xsems = (sem_x0, sem_x1, sem_x2)

    trows = MAX_DEG // NS
    tr = pl.ds(sid * trows, trows)
    ct1 = pltpu.async_copy(zin_hbm.at[tr], zin_sh.at[tr], sem_t)
    ct2 = pltpu.async_copy(zout_hbm.at[tr], zout_sh.at[tr], sem_t)

    for k in range(2):
        ck = wid + k * NW

        @pl.when(ck < NFULL)
        def _():
            pltpu.async_copy(x_hbm.at[pl.ds(ck * CHUNK, CHUNK)],
                             xacc_v.at[k], xsems[k])

    ebase = sid * EPT
    ce = pltpu.async_copy(edges_hbm.at[pl.ds(N_EDGES + ebase, ECHUNK)],
                          ebuf0_v, sem_e)

    zeros = jnp.zeros((L,), jnp.int32)

    @plsc.parallel_loop(0, HROWS, unroll=4)
    def _(r):
        for j in range(HCOLS // L):
            s = pl.ds(j * L, L)
            hin_v[r, s] = zeros
            hout_v[r, s] = zeros

    iota = lax.iota(jnp.int32, L)
    for i in range(HROWS // L):
        idx80_v[pl.ds(i * L, L)] = iota + (i * L)

    @pl.when(sid == 0)
    def _():
        pltpu.sync_copy(hin_v, degin_sh)
        pltpu.sync_copy(hout_v, degout_sh)
    plsc.subcore_barrier()

    ones = jnp.ones((L,), jnp.int32)

    def scatter_chunk(ebuf, hist):
        @plsc.parallel_loop(0, ECHUNK, step=L, unroll=SC_UNROLL)
        def _(i):
            idx = ebuf[pl.ds(i, L)]
            plsc.addupdate_scatter(
                hist,
                [lax.shift_right_logical(idx, 7),
                 lax.bitwise_and(idx, 127)],
                ones)

    ce.wait()
    ce = pltpu.async_copy(
        edges_hbm.at[pl.ds(N_EDGES + ebase + ECHUNK, ECHUNK)], ebuf1_v,
        sem_e)
    scatter_chunk(ebuf0_v, hin_v)
    ce.wait()
    ce = pltpu.async_copy(edges_hbm.at[pl.ds(ebase, ECHUNK)], ebuf0_v,
                          sem_e)
    scatter_chunk(ebuf1_v, hin_v)
    ce.wait()
    ce = pltpu.async_copy(edges_hbm.at[pl.ds(ebase + ECHUNK, ECHUNK)],
                          ebuf1_v, sem_e)
    scatter_chunk(ebuf0_v, hout_v)
    ce.wait()
    scatter_chunk(ebuf1_v, hout_v)

    pltpu.sync_copy(hin_v, degin_sh.at[idx80_v], add=True)
    pltpu.sync_copy(hout_v, degout_sh.at[idx80_v], add=True)
    ct1.wait()
    ct2.wait()
    plsc.subcore_barrier()

    cap = jnp.full((L,), MAX_DEG - 1, jnp.int32)

    def load_deg(row, di, do):
        pltpu.sync_copy(degin_sh.at[row], di)
        pltpu.sync_copy(degout_sh.at[row], do)
        for j in range(CHUNK // L):
            s = pl.ds(j * L, L)
            di[s] = jnp.minimum(di[s], cap)
            do[s] = jnp.minimum(do[s], cap)

    def add_rows(k, nrows):
        @plsc.parallel_loop(0, nrows, unroll=ROW_UNROLL)
        def _(r):
            for j in range(NODE_DIM // L):
                s = pl.ds(j * L, L)
                plsc.addupdate(xacc_v.at[k, r, s], a_v[r, s] + b_v[r, s])

    for k in range(KMAX):
        ck = wid + k * NW
        slot = k % 2

        @pl.when(ck < NFULL)
        def _():
            nb = ck * CHUNK
            load_deg(ck, di_refs[k], do_refs[k])
            ga = pltpu.async_copy(zin_sh.at[di_refs[k]], a_v, sem_a)
            gb = pltpu.async_copy(zout_sh.at[do_refs[k]], b_v, sem_b)
            if k == 2:
                pltpu.make_async_copy(
                    xacc_v.at[0], out_hbm.at[pl.ds(wid * CHUNK, CHUNK)],
                    sem_o).wait()
                pltpu.async_copy(x_hbm.at[pl.ds(nb, CHUNK)],
                                 xacc_v.at[slot], xsems[0])
            pltpu.make_async_copy(x_hbm.at[pl.ds(nb, CHUNK)],
                                  xacc_v.at[slot], xsems[0 if k == 2 else k]
                                  ).wait()
            ga.wait()
            gb.wait()
            add_rows(slot, CHUNK)
            pltpu.async_copy(xacc_v.at[slot], out_hbm.at[pl.ds(nb, CHUNK)],
                             sem_o)

    @pl.when(wid == NW - 1)
    def _():
        load_deg(NFULL, di_refs[2], do_refs[2])
        ga = pltpu.async_copy(zin_sh.at[di_refs[2]], a_v, sem_a)
        gb = pltpu.async_copy(zout_sh.at[do_refs[2]], b_v, sem_b)
        pltpu.make_async_copy(
            xacc_v.at[0], out_hbm.at[pl.ds(wid * CHUNK, CHUNK)],
            sem_o).wait()
        pltpu.async_copy(x_hbm.at[pl.ds(TBASE, TAIL)],
                         xacc_v.at[0, pl.ds(0, TAIL)], sem_x2)
        pltpu.make_async_copy(x_hbm.at[pl.ds(TBASE, TAIL)],
                              xacc_v.at[0, pl.ds(0, TAIL)], sem_x2).wait()
        ga.wait()
        gb.wait()
        for i in range(TAIL):
            for j in range(NODE_DIM // L):
                s = pl.ds(j * L, L)
                xacc_v[0, i, s] = xacc_v[0, i, s] + a_v[i, s] + b_v[i, s]
        pltpu.sync_copy(xacc_v.at[0, pl.ds(0, TAIL)],
                        out_hbm.at[pl.ds(TBASE, TAIL)])

    skip0 = ((wid + 2 * NW) < NFULL) | (wid == NW - 1)
    for k in range(KMAX):
        ck = wid + k * NW
        cond = (ck < NFULL) & (~skip0) if k == 0 else (ck < NFULL)

        @pl.when(cond)
        def _():
            pltpu.make_async_copy(
                xacc_v.at[k % 2], out_hbm.at[pl.ds(ck * CHUNK, CHUNK)],
                sem_o).wait()


def kernel(x, edge_index, z_in, z_out):
    edges = edge_index.astype(jnp.int32).reshape(-1)

    call = pl.kernel(
        _fused_kernel,
        out_type=jax.ShapeDtypeStruct((N_NODES, NODE_DIM), jnp.float32),
        mesh=_mesh,
        scratch_types=[
            pltpu.VMEM((ECHUNK,), jnp.int32),
            pltpu.VMEM((ECHUNK,), jnp.int32),
            pltpu.VMEM((HROWS, HCOLS), jnp.int32),
            pltpu.VMEM((HROWS, HCOLS), jnp.int32),
            pltpu.VMEM((HROWS,), jnp.int32),
            pltpu.VMEM((CHUNK,), jnp.int32),
            pltpu.VMEM((CHUNK,), jnp.int32),
            pltpu.VMEM((CHUNK,), jnp.int32),
            pltpu.VMEM((CHUNK,), jnp.int32),
            pltpu.VMEM((CHUNK,), jnp.int32),
            pltpu.VMEM((CHUNK,), jnp.int32),
            pltpu.VMEM((2, CHUNK, NODE_DIM), jnp.float32),
            pltpu.VMEM((CHUNK, NODE_DIM), jnp.float32),
            pltpu.VMEM((CHUNK, NODE_DIM), jnp.float32),
            pltpu.VMEM_SHARED((MAX_DEG, NODE_DIM), jnp.float32),
            pltpu.VMEM_SHARED((MAX_DEG, NODE_DIM), jnp.float32),
            pltpu.VMEM_SHARED((HROWS, HCOLS), jnp.int32),
            pltpu.VMEM_SHARED((HROWS, HCOLS), jnp.int32),
            pltpu.SemaphoreType.DMA,
            pltpu.SemaphoreType.DMA,
            pltpu.SemaphoreType.DMA,
            pltpu.SemaphoreType.DMA,
            pltpu.SemaphoreType.DMA,
            pltpu.SemaphoreType.DMA,
            pltpu.SemaphoreType.DMA,
            pltpu.SemaphoreType.DMA,
        ],
        compiler_params=pltpu.CompilerParams(needs_layout_passes=False),
    )
    return call(x, edges, z_in, z_out)
